# Initial kernel scaffold; baseline (speedup 1.0000x reference)
import jax
import jax.numpy as jnp
from jax.experimental import pallas as pl


def kernel(indices, weight):
    raise NotImplementedError("write your pallas kernel here")


# speedup vs baseline: 1.7686x; 1.7686x over previous
import jax
import jax.numpy as jnp
from jax.experimental import pallas as pl
from jax.experimental.pallas import tpu as pltpu

_TOKEN_TILE = 512


def _gather_kernel(idx_ref, w_hbm, out_ref, sem):
    tm = out_ref.shape[0]
    base = pl.program_id(0) * tm

    for r in range(tm):
        row = idx_ref[base + r]
        pltpu.make_async_copy(
            w_hbm.at[pl.ds(row, 1), :],
            out_ref.at[pl.ds(r, 1), :],
            sem,
        ).start()

    pltpu.make_async_copy(
        w_hbm.at[pl.ds(0, tm), :],
        out_ref.at[pl.ds(0, tm), :],
        sem,
    ).wait()


def kernel(indices, weight):
    num_embeddings, embedding_dim = weight.shape
    orig_shape = indices.shape
    flat_idx = indices.reshape(-1)
    if flat_idx.dtype != jnp.int32:
        flat_idx = flat_idx.astype(jnp.int32)
    n = flat_idx.shape[0]
    if n == 0:
        return jnp.zeros(orig_shape + (embedding_dim,), weight.dtype)

    tm = min(_TOKEN_TILE, n) if n % _TOKEN_TILE else _TOKEN_TILE
    n_pad = -(-n // tm) * tm
    if n_pad != n:
        flat_idx = jnp.pad(flat_idx, (0, n_pad - n))
    n_tiles = n_pad // tm

    grid_spec = pltpu.PrefetchScalarGridSpec(
        num_scalar_prefetch=1,
        grid=(n_tiles,),
        in_specs=[pl.BlockSpec(memory_space=pl.ANY)],
        out_specs=pl.BlockSpec((tm, embedding_dim), lambda i, idx: (i, 0)),
        scratch_shapes=[pltpu.SemaphoreType.DMA],
    )
    flat_out = pl.pallas_call(
        _gather_kernel,
        out_shape=jax.ShapeDtypeStruct((n_pad, embedding_dim), weight.dtype),
        grid_spec=grid_spec,
        compiler_params=pltpu.CompilerParams(
            dimension_semantics=("parallel",),
            disable_bounds_checks=True,
        ),
    )(flat_idx, weight)
    if n_pad != n:
        flat_out = flat_out[:n]
    return flat_out.reshape(orig_shape + (embedding_dim,))


# speedup vs baseline: 2.0401x; 1.1535x over previous
import jax
import jax.numpy as jnp
from jax.experimental import pallas as pl
from jax.experimental.pallas import tpu as pltpu

_TOKEN_TILE = 1024


def _gather_kernel(idx_ref, w_hbm, out_ref, sem):
    tm = out_ref.shape[0]
    base = pl.program_id(0) * tm

    for r in range(tm):
        row = idx_ref[base + r]
        pltpu.make_async_copy(
            w_hbm.at[pl.ds(row, 1), :],
            out_ref.at[pl.ds(r, 1), :],
            sem,
        ).start()

    pltpu.make_async_copy(
        w_hbm.at[pl.ds(0, tm), :],
        out_ref.at[pl.ds(0, tm), :],
        sem,
    ).wait()


def kernel(indices, weight):
    num_embeddings, embedding_dim = weight.shape
    orig_shape = indices.shape
    flat_idx = indices.reshape(-1)
    if flat_idx.dtype != jnp.int32:
        flat_idx = flat_idx.astype(jnp.int32)
    n = flat_idx.shape[0]
    if n == 0:
        return jnp.zeros(orig_shape + (embedding_dim,), weight.dtype)

    tm = min(_TOKEN_TILE, n) if n % _TOKEN_TILE else _TOKEN_TILE
    n_pad = -(-n // tm) * tm
    if n_pad != n:
        flat_idx = jnp.pad(flat_idx, (0, n_pad - n))
    n_tiles = n_pad // tm

    grid_spec = pltpu.PrefetchScalarGridSpec(
        num_scalar_prefetch=1,
        grid=(n_tiles,),
        in_specs=[pl.BlockSpec(memory_space=pl.ANY)],
        out_specs=pl.BlockSpec((tm, embedding_dim), lambda i, idx: (i, 0)),
        scratch_shapes=[pltpu.SemaphoreType.DMA],
    )
    flat_out = pl.pallas_call(
        _gather_kernel,
        out_shape=jax.ShapeDtypeStruct((n_pad, embedding_dim), weight.dtype),
        grid_spec=grid_spec,
        compiler_params=pltpu.CompilerParams(
            dimension_semantics=("parallel",),
            disable_bounds_checks=True,
        ),
    )(flat_idx, weight)
    if n_pad != n:
        flat_out = flat_out[:n]
    return flat_out.reshape(orig_shape + (embedding_dim,))


# speedup vs baseline: 2.1960x; 1.0764x over previous
import jax
import jax.numpy as jnp
from jax.experimental import pallas as pl
from jax.experimental.pallas import tpu as pltpu

_TOKEN_TILE = 2048


def _gather_kernel(idx_ref, w_hbm, out_ref, sem):
    tm = out_ref.shape[0]
    base = pl.program_id(0) * tm

    for r in range(tm):
        row = idx_ref[base + r]
        pltpu.make_async_copy(
            w_hbm.at[pl.ds(row, 1), :],
            out_ref.at[pl.ds(r, 1), :],
            sem,
        ).start()

    pltpu.make_async_copy(
        w_hbm.at[pl.ds(0, tm), :],
        out_ref.at[pl.ds(0, tm), :],
        sem,
    ).wait()


def kernel(indices, weight):
    num_embeddings, embedding_dim = weight.shape
    orig_shape = indices.shape
    flat_idx = indices.reshape(-1)
    if flat_idx.dtype != jnp.int32:
        flat_idx = flat_idx.astype(jnp.int32)
    n = flat_idx.shape[0]
    if n == 0:
        return jnp.zeros(orig_shape + (embedding_dim,), weight.dtype)

    tm = min(_TOKEN_TILE, n) if n % _TOKEN_TILE else _TOKEN_TILE
    n_pad = -(-n // tm) * tm
    if n_pad != n:
        flat_idx = jnp.pad(flat_idx, (0, n_pad - n))
    n_tiles = n_pad // tm

    grid_spec = pltpu.PrefetchScalarGridSpec(
        num_scalar_prefetch=1,
        grid=(n_tiles,),
        in_specs=[pl.BlockSpec(memory_space=pl.ANY)],
        out_specs=pl.BlockSpec((tm, embedding_dim), lambda i, idx: (i, 0)),
        scratch_shapes=[pltpu.SemaphoreType.DMA],
    )
    flat_out = pl.pallas_call(
        _gather_kernel,
        out_shape=jax.ShapeDtypeStruct((n_pad, embedding_dim), weight.dtype),
        grid_spec=grid_spec,
        compiler_params=pltpu.CompilerParams(
            dimension_semantics=("parallel",),
            disable_bounds_checks=True,
        ),
    )(flat_idx, weight)
    if n_pad != n:
        flat_out = flat_out[:n]
    return flat_out.reshape(orig_shape + (embedding_dim,))


# speedup vs baseline: 2.2505x; 1.0249x over previous
import jax
import jax.numpy as jnp
from jax.experimental import pallas as pl
from jax.experimental.pallas import tpu as pltpu

_TOKEN_TILE = 4096


def _gather_kernel(idx_ref, w_hbm, out_ref, sem):
    tm = out_ref.shape[0]
    base = pl.program_id(0) * tm

    for r in range(tm):
        row = idx_ref[base + r]
        pltpu.make_async_copy(
            w_hbm.at[pl.ds(row, 1), :],
            out_ref.at[pl.ds(r, 1), :],
            sem,
        ).start()

    pltpu.make_async_copy(
        w_hbm.at[pl.ds(0, tm), :],
        out_ref.at[pl.ds(0, tm), :],
        sem,
    ).wait()


def kernel(indices, weight):
    num_embeddings, embedding_dim = weight.shape
    orig_shape = indices.shape
    flat_idx = indices.reshape(-1)
    if flat_idx.dtype != jnp.int32:
        flat_idx = flat_idx.astype(jnp.int32)
    n = flat_idx.shape[0]
    if n == 0:
        return jnp.zeros(orig_shape + (embedding_dim,), weight.dtype)

    tm = min(_TOKEN_TILE, n) if n % _TOKEN_TILE else _TOKEN_TILE
    n_pad = -(-n // tm) * tm
    if n_pad != n:
        flat_idx = jnp.pad(flat_idx, (0, n_pad - n))
    n_tiles = n_pad // tm

    grid_spec = pltpu.PrefetchScalarGridSpec(
        num_scalar_prefetch=1,
        grid=(n_tiles,),
        in_specs=[pl.BlockSpec(memory_space=pl.ANY)],
        out_specs=pl.BlockSpec((tm, embedding_dim), lambda i, idx: (i, 0)),
        scratch_shapes=[pltpu.SemaphoreType.DMA],
    )
    flat_out = pl.pallas_call(
        _gather_kernel,
        out_shape=jax.ShapeDtypeStruct((n_pad, embedding_dim), weight.dtype),
        grid_spec=grid_spec,
        compiler_params=pltpu.CompilerParams(
            dimension_semantics=("parallel",),
            disable_bounds_checks=True,
        ),
    )(flat_idx, weight)
    if n_pad != n:
        flat_out = flat_out[:n]
    return flat_out.reshape(orig_shape + (embedding_dim,))


# speedup vs baseline: 2.5844x; 1.1484x over previous
import jax
import jax.numpy as jnp
from jax.experimental import pallas as pl
from jax.experimental.pallas import tpu as pltpu

_TOKEN_TILE = 4096


def _gather_kernel(idx_ref, w_hbm, out_ref, sem):
    tm = out_ref.shape[0]
    base = pl.program_id(0) * tm

    for r in range(tm):
        row = idx_ref[base + r]
        pltpu.make_async_copy(
            w_hbm.at[pl.ds(row, 1), :],
            out_ref.at[pl.ds(r, 1), :],
            sem,
        ).start(priority=r & 1)

    pltpu.make_async_copy(
        w_hbm.at[pl.ds(0, tm), :],
        out_ref.at[pl.ds(0, tm), :],
        sem,
    ).wait()


def kernel(indices, weight):
    num_embeddings, embedding_dim = weight.shape
    orig_shape = indices.shape
    flat_idx = indices.reshape(-1)
    if flat_idx.dtype != jnp.int32:
        flat_idx = flat_idx.astype(jnp.int32)
    n = flat_idx.shape[0]
    if n == 0:
        return jnp.zeros(orig_shape + (embedding_dim,), weight.dtype)

    tm = min(_TOKEN_TILE, n) if n % _TOKEN_TILE else _TOKEN_TILE
    n_pad = -(-n // tm) * tm
    if n_pad != n:
        flat_idx = jnp.pad(flat_idx, (0, n_pad - n))
    n_tiles = n_pad // tm

    grid_spec = pltpu.PrefetchScalarGridSpec(
        num_scalar_prefetch=1,
        grid=(n_tiles,),
        in_specs=[pl.BlockSpec(memory_space=pl.ANY)],
        out_specs=pl.BlockSpec((tm, embedding_dim), lambda i, idx: (i, 0)),
        scratch_shapes=[pltpu.SemaphoreType.DMA],
    )
    flat_out = pl.pallas_call(
        _gather_kernel,
        out_shape=jax.ShapeDtypeStruct((n_pad, embedding_dim), weight.dtype),
        grid_spec=grid_spec,
        compiler_params=pltpu.CompilerParams(
            dimension_semantics=("parallel",),
            disable_bounds_checks=True,
        ),
    )(flat_idx, weight)
    if n_pad != n:
        flat_out = flat_out[:n]
    return flat_out.reshape(orig_shape + (embedding_dim,))
